# Initial kernel scaffold; baseline (speedup 1.0000x reference)
import jax
import jax.numpy as jnp
from jax.experimental import pallas as pl


def kernel(features, edge_index, W1, b1, W2, b2, Wp, bp):
    raise NotImplementedError("write your pallas kernel here")


# speedup vs baseline: 3.8537x; 3.8537x over previous
import functools

import jax
import jax.numpy as jnp
from jax import lax
from jax.experimental import pallas as pl
from jax.experimental.pallas import tpu as pltpu
from jax.experimental.pallas import tpu_sc as plsc

N = 10000
E = 320000
D = 128
NCLS = 16

NC = 2
NS = 16
NW = NC * NS
NPAD = 10112
RPT = NPAD // NS
ET = E // NW
CH = 128
NCHUNK = 80
ETP = NCHUNK * CH
NHIST = 2 * NPAD
HPT = NHIST // NS
HW = 16

_mesh = plsc.VectorSubcoreMesh(core_axis_name="c", subcore_axis_name="s")
_sc_params = pltpu.CompilerParams(use_tc_tiling_on_sc=False)


@functools.partial(
    pl.kernel,
    mesh=_mesh,
    out_type=jax.ShapeDtypeStruct((NC, NHIST, HW), jnp.float32),
    scratch_types=[
        pltpu.VMEM((2 * NCHUNK, CH), jnp.int32),
        pltpu.VMEM((CH, HW), jnp.float32),
        pltpu.VMEM((HPT, HW), jnp.float32),
        pltpu.VMEM_SHARED((NHIST, HW), jnp.float32),
    ],
    compiler_params=_sc_params,
)
def _sc_degrees(idx_hbm, out_hbm, idx_v, ones_v, zb_v, deg_sp):
    cid = lax.axis_index("c")
    sid = lax.axis_index("s")
    t = cid * NS + sid

    def _ones(i, _):
        ones_v[i, :] = jnp.full((HW,), 1.0, jnp.float32)
        return 0

    lax.fori_loop(0, CH, _ones, 0)

    def _zb(i, _):
        zb_v[i, :] = jnp.zeros((HW,), jnp.float32)
        return 0

    lax.fori_loop(0, HPT, _zb, 0)
    pltpu.sync_copy(zb_v, deg_sp.at[pl.ds(sid * HPT, HPT)])
    plsc.subcore_barrier()

    pltpu.sync_copy(idx_hbm.at[t], idx_v)

    def _step(j, _):
        pltpu.sync_copy(ones_v, deg_sp.at[idx_v.at[j]], add=True)
        return 0

    lax.fori_loop(0, 2 * NCHUNK, _step, 0)
    plsc.subcore_barrier()
    pltpu.sync_copy(deg_sp.at[pl.ds(sid * HPT, HPT)],
                    out_hbm.at[cid, pl.ds(sid * HPT, HPT)])


@functools.partial(
    pl.kernel,
    mesh=_mesh,
    out_type=jax.ShapeDtypeStruct((NC, NPAD, D), jnp.float32),
    scratch_types=[
        pltpu.VMEM((NCHUNK, CH), jnp.int32),
        pltpu.VMEM((NCHUNK, CH), jnp.int32),
        pltpu.VMEM((CH, D), jnp.float32),
        pltpu.VMEM_SHARED((NPAD, D), jnp.float32),
        pltpu.SemaphoreType.DMA,
    ],
    compiler_params=_sc_params,
)
def _sc_scatter(h_hbm, src_hbm, dst_hbm, out_hbm, src_v, dst_v, rows_v, agg_sp, sem):
    cid = lax.axis_index("c")
    sid = lax.axis_index("s")
    t = cid * NS + sid

    def _zrow(i, _):
        def _zcol(j, _):
            rows_v[i, pl.ds(j * 16, 16)] = jnp.zeros((16,), jnp.float32)
            return 0

        lax.fori_loop(0, D // 16, _zcol, 0)
        return 0

    lax.fori_loop(0, CH, _zrow, 0)
    base = sid * RPT
    for k in range(RPT // CH):
        pltpu.sync_copy(rows_v, agg_sp.at[pl.ds(base + k * CH, CH)])
    rem = RPT % CH
    pltpu.sync_copy(rows_v.at[pl.ds(0, rem)],
                    agg_sp.at[pl.ds(base + (RPT // CH) * CH, rem)])
    plsc.subcore_barrier()

    pltpu.sync_copy(src_hbm.at[t], src_v)
    pltpu.sync_copy(dst_hbm.at[t], dst_v)

    def _step(j, _):
        pltpu.async_copy(h_hbm.at[src_v.at[j]], rows_v, sem).wait()
        pltpu.sync_copy(rows_v, agg_sp.at[dst_v.at[j]], add=True)
        return 0

    lax.fori_loop(0, NCHUNK, _step, 0)
    plsc.subcore_barrier()
    for k in range(RPT // CH):
        pltpu.sync_copy(agg_sp.at[pl.ds(base + k * CH, CH)],
                        out_hbm.at[cid, pl.ds(base + k * CH, CH)])
    pltpu.sync_copy(agg_sp.at[pl.ds(base + (RPT // CH) * CH, rem)],
                    out_hbm.at[cid, pl.ds(base + (RPT // CH) * CH, rem)])


def _tc_layer1(x_ref, w_ref, deg_ref, o_ref):
    d = deg_ref[...]
    ns = lax.rsqrt(jnp.maximum((d[0, 0] + d[1, 0])[:, 0:1], 1.0))
    h = jnp.dot(x_ref[...], w_ref[...], preferred_element_type=jnp.float32)
    o_ref[...] = h * ns


def _tc_layer2(a_ref, deg_ref, b_ref, w_ref, o_ref):
    d = deg_ref[...]
    ns = lax.rsqrt(jnp.maximum((d[0, 0] + d[1, 0])[:, 0:1], 1.0))
    nd = lax.rsqrt(jnp.maximum((d[0, 1] + d[1, 1])[:, 0:1], 1.0))
    t = jnp.maximum((a_ref[0] + a_ref[1]) * nd + b_ref[...], 0.0)
    o_ref[...] = jnp.dot(t, w_ref[...], preferred_element_type=jnp.float32) * ns


def _tc_head(a_ref, deg_ref, b_ref, wp_ref, bp_ref, o_ref):
    d = deg_ref[...]
    nd = lax.rsqrt(jnp.maximum((d[0, 1] + d[1, 1])[:, 0:1], 1.0))
    t = jnp.maximum((a_ref[0] + a_ref[1]) * nd + b_ref[...], 0.0)
    s = jnp.sum(t[:N, :], axis=0, keepdims=True) * (1.0 / N)
    o_ref[...] = jnp.dot(s, wp_ref[...], preferred_element_type=jnp.float32) + bp_ref[...]


_tc1 = pl.pallas_call(
    _tc_layer1, out_shape=jax.ShapeDtypeStruct((NPAD, D), jnp.float32))
_tc2 = pl.pallas_call(
    _tc_layer2, out_shape=jax.ShapeDtypeStruct((NPAD, D), jnp.float32))
_tc3 = pl.pallas_call(
    _tc_head, out_shape=jax.ShapeDtypeStruct((1, NCLS), jnp.float32))


def kernel(features, edge_index, W1, b1, W2, b2, Wp, bp):
    src = edge_index[0].astype(jnp.int32).reshape(NW, ET)
    dst = edge_index[1].astype(jnp.int32).reshape(NW, ET)
    pad = jnp.full((NW, ETP - ET), N, jnp.int32)
    src_p = jnp.concatenate([src, pad], axis=1).reshape(NW, NCHUNK, CH)
    dst_p = jnp.concatenate([dst, pad], axis=1).reshape(NW, NCHUNK, CH)
    hist_idx = jnp.concatenate([src_p, dst_p + NPAD], axis=1)

    x_pad = jnp.pad(features, ((0, NPAD - N), (0, 0)))

    degh = _sc_degrees(hist_idx)
    deg4 = degh.reshape(NC, 2, NPAD, HW)

    h1 = _tc1(x_pad, W1, deg4)
    agg1 = _sc_scatter(h1, src_p, dst_p)
    h2 = _tc2(agg1, deg4, b1.reshape(1, D), W2)
    agg2 = _sc_scatter(h2, src_p, dst_p)
    return _tc3(agg2, deg4, b2.reshape(1, D), Wp, bp.reshape(1, NCLS))
